# Initial kernel scaffold; baseline (speedup 1.0000x reference)
import jax
import jax.numpy as jnp
from jax.experimental import pallas as pl


def kernel(A_rows, A_cols, A_values, b, c, x, Iy):
    raise NotImplementedError("write your pallas kernel here")


# speedup vs baseline: 166.6562x; 166.6562x over previous
import functools

import jax
import jax.numpy as jnp
from jax import lax
from jax.experimental import pallas as pl
from jax.experimental.pallas import tpu as pltpu
from jax.experimental.pallas import tpu_sc as plsc

M = 65536
N = 65536
NNZ = 4194304

NC = 2
NS = 16
NW = NC * NS
CHUNK = NNZ // NW
BLK = 4096
NBLK = CHUNK // BLK
ROWS_PER_SUB = M // NS


def _spmv_body(rows_hbm, cols_hbm, vals_hbm, x_hbm, out_hbm,
               x_v, cols_v, rows_v, vals_v, prod_v, zero_v, acc_sh):
    c = lax.axis_index("c")
    s = lax.axis_index("s")
    wid = s * NC + c

    def _zero(i, carry):
        zero_v[pl.ds(i * 16, 16)] = jnp.zeros((16,), jnp.float32)
        return carry
    lax.fori_loop(0, ROWS_PER_SUB // 16, _zero, 0)
    pltpu.sync_copy(zero_v, acc_sh.at[pl.ds(s * ROWS_PER_SUB, ROWS_PER_SUB)])

    pltpu.sync_copy(x_hbm, x_v)
    plsc.subcore_barrier()

    base = wid * CHUNK

    def _block(blk, carry):
        off = pl.multiple_of(base + blk * BLK, BLK)
        roff = pl.multiple_of((base + blk * BLK) // 128, BLK // 128)
        pltpu.sync_copy(cols_hbm.at[pl.ds(off, BLK)], cols_v)
        pltpu.sync_copy(vals_hbm.at[pl.ds(off, BLK)], vals_v)
        pltpu.sync_copy(rows_hbm.at[pl.ds(roff, BLK // 128)], rows_v)

        def _inner(i, icarry):
            cols16 = cols_v[pl.ds(i * 16, 16)]
            vals16 = vals_v[pl.ds(i * 16, 16)]
            xg = plsc.load_gather(x_v, [cols16])
            prod_v[pl.ds(i * 16, 16)] = vals16 * xg
            return icarry
        lax.fori_loop(0, BLK // 16, _inner, 0)

        def _scat(j, jcarry):
            pltpu.sync_copy(prod_v.at[pl.ds(j * 128, 128)],
                            acc_sh.at[rows_v.at[j]], add=True)
            return jcarry
        lax.fori_loop(0, BLK // 128, _scat, 0)
        return carry

    lax.fori_loop(0, NBLK, _block, 0)

    plsc.subcore_barrier()
    pltpu.sync_copy(acc_sh.at[pl.ds(s * ROWS_PER_SUB, ROWS_PER_SUB)],
                    out_hbm.at[c, pl.ds(s * ROWS_PER_SUB, ROWS_PER_SUB)])


_spmv = functools.partial(
    pl.kernel,
    out_type=jax.ShapeDtypeStruct((NC, M), jnp.float32),
    mesh=plsc.VectorSubcoreMesh(core_axis_name="c", subcore_axis_name="s"),
    compiler_params=pltpu.CompilerParams(needs_layout_passes=False),
    scratch_types=[
        pltpu.VMEM((N,), jnp.float32),
        pltpu.VMEM((BLK,), jnp.int32),
        pltpu.VMEM((BLK // 128, 128), jnp.int32),
        pltpu.VMEM((BLK,), jnp.float32),
        pltpu.VMEM((BLK,), jnp.float32),
        pltpu.VMEM((ROWS_PER_SUB,), jnp.float32),
        pltpu.VMEM_SHARED((M,), jnp.float32),
    ],
)(_spmv_body)


def _finish_body(acc_ref, b_ref, iy_ref, out_ref):
    y = acc_ref[0] + acc_ref[1] - b_ref[...]
    py = y + iy_ref[...] * jnp.maximum(-y, 0.0)
    part_2 = jnp.max(jnp.abs(py))
    part_3 = 1.0 + jnp.max(jnp.abs(b_ref[...]))
    out_ref[0, 0] = part_2 / part_3


_finish = pl.pallas_call(
    _finish_body,
    out_shape=jax.ShapeDtypeStruct((1, 1), jnp.float32),
    out_specs=pl.BlockSpec(memory_space=pltpu.SMEM),
)


def kernel(A_rows, A_cols, A_values, b, c, x, Iy):
    rows2d = A_rows.astype(jnp.int32).reshape(NNZ // 128, 128)
    cols = A_cols.astype(jnp.int32)
    xf = x[:, 0]
    acc = _spmv(rows2d, cols, A_values, xf)
    out = _finish(acc.reshape(NC, 512, 128),
                  b.reshape(512, 128),
                  Iy.reshape(512, 128))
    return out[0, 0]


# speedup vs baseline: 289.3090x; 1.7360x over previous
import functools

import jax
import jax.numpy as jnp
from jax import lax
from jax.experimental import pallas as pl
from jax.experimental.pallas import tpu as pltpu
from jax.experimental.pallas import tpu_sc as plsc

M = 65536
N = 65536
NNZ = 4194304

NC = 2
NS = 16
NW = NC * NS
CHUNK = NNZ // NW
BLK = 4096
NBLK = CHUNK // BLK
NSEG = BLK // 128
ROWS_PER_SUB = M // NS


def _spmv_body(rows_hbm, cols_hbm, vals_hbm, x_hbm, out_hbm,
               x_v, cols_v, vals_v, rows_v, prod_v, zero_v, acc_sh,
               x_sem, in_sems, rows_sems, scat_sems):
    c = lax.axis_index("c")
    s = lax.axis_index("s")
    wid = s * NC + c
    base = wid * CHUNK
    rbase = base // 128

    pltpu.async_copy(x_hbm, x_v, x_sem)

    def _zero(i, carry):
        zero_v[pl.ds(i * 16, 16)] = jnp.zeros((16,), jnp.float32)
        return carry
    lax.fori_loop(0, ROWS_PER_SUB // 16, _zero, 0)
    pltpu.sync_copy(zero_v, acc_sh.at[pl.ds(s * ROWS_PER_SUB, ROWS_PER_SUB)])

    pltpu.make_async_copy(x_hbm, x_v, x_sem).wait()
    plsc.subcore_barrier()

    def start_inputs(blk, b, r):
        off = pl.multiple_of(base + blk * BLK, BLK)
        roff = pl.multiple_of(rbase + blk * NSEG, NSEG)
        pltpu.async_copy(cols_hbm.at[pl.ds(off, BLK)], cols_v.at[b],
                         in_sems.at[b, 0])
        pltpu.async_copy(vals_hbm.at[pl.ds(off, BLK)], vals_v.at[b],
                         in_sems.at[b, 1])
        pltpu.async_copy(rows_hbm.at[pl.ds(roff, NSEG)], rows_v.at[r],
                         rows_sems.at[r])

    def wait_inputs(b, r):
        pltpu.make_async_copy(cols_hbm.at[pl.ds(0, BLK)], cols_v.at[b],
                              in_sems.at[b, 0]).wait()
        pltpu.make_async_copy(vals_hbm.at[pl.ds(0, BLK)], vals_v.at[b],
                              in_sems.at[b, 1]).wait()
        pltpu.make_async_copy(rows_hbm.at[pl.ds(0, NSEG)], rows_v.at[r],
                              rows_sems.at[r]).wait()

    def compute(b):
        def _inner(i, icarry):
            cols16 = cols_v[b, pl.ds(i * 16, 16)]
            vals16 = vals_v[b, pl.ds(i * 16, 16)]
            xg = plsc.load_gather(x_v, [cols16])
            prod_v[b, pl.ds(i * 16, 16)] = vals16 * xg
            return icarry
        lax.fori_loop(0, BLK // 16, _inner, 0)

    def fire_scatters(b, r):
        def _fire(j, jcarry):
            pltpu.async_copy(prod_v.at[b, pl.ds(j * 128, 128)],
                             acc_sh.at[rows_v.at[r, j]],
                             scat_sems.at[b], add=True)
            return jcarry
        lax.fori_loop(0, NSEG, _fire, 0)

    def drain_scatters(b, r):
        def _drain(j, jcarry):
            pltpu.make_async_copy(prod_v.at[b, pl.ds(j * 128, 128)],
                                  acc_sh.at[rows_v.at[r, j]],
                                  scat_sems.at[b]).wait()
            return jcarry
        lax.fori_loop(0, NSEG, _drain, 0)

    start_inputs(0, 0, 0)

    def _quad(q, carry):
        for u in range(4):
            blk = q * 4 + u
            b = u % 2
            r = u
            nxt_ok = blk + 1 < NBLK

            @pl.when(nxt_ok)
            def _():
                start_inputs(blk + 1, 1 - b, (u + 1) % 4)

            wait_inputs(b, r)

            @pl.when(blk >= 2)
            def _():
                drain_scatters(b, (u + 2) % 4)

            compute(b)
            fire_scatters(b, r)
        return carry

    lax.fori_loop(0, NBLK // 4, _quad, 0)

    drain_scatters(0, 2)
    drain_scatters(1, 3)

    plsc.subcore_barrier()
    pltpu.sync_copy(acc_sh.at[pl.ds(s * ROWS_PER_SUB, ROWS_PER_SUB)],
                    out_hbm.at[c, pl.ds(s * ROWS_PER_SUB, ROWS_PER_SUB)])


_spmv = functools.partial(
    pl.kernel,
    out_type=jax.ShapeDtypeStruct((NC, M), jnp.float32),
    mesh=plsc.VectorSubcoreMesh(core_axis_name="c", subcore_axis_name="s"),
    compiler_params=pltpu.CompilerParams(needs_layout_passes=False),
    scratch_types=[
        pltpu.VMEM((N,), jnp.float32),
        pltpu.VMEM((2, BLK), jnp.int32),
        pltpu.VMEM((2, BLK), jnp.float32),
        pltpu.VMEM((4, NSEG, 128), jnp.int32),
        pltpu.VMEM((2, BLK), jnp.float32),
        pltpu.VMEM((ROWS_PER_SUB,), jnp.float32),
        pltpu.VMEM_SHARED((M,), jnp.float32),
        pltpu.SemaphoreType.DMA,
        pltpu.SemaphoreType.DMA((2, 2)),
        pltpu.SemaphoreType.DMA((4,)),
        pltpu.SemaphoreType.DMA((2,)),
    ],
)(_spmv_body)


def _finish_body(acc_ref, b_ref, iy_ref, out_ref):
    y = acc_ref[0] + acc_ref[1] - b_ref[...]
    py = y + iy_ref[...] * jnp.maximum(-y, 0.0)
    part_2 = jnp.max(jnp.abs(py))
    part_3 = 1.0 + jnp.max(jnp.abs(b_ref[...]))
    out_ref[0, 0] = part_2 / part_3


_finish = pl.pallas_call(
    _finish_body,
    out_shape=jax.ShapeDtypeStruct((1, 1), jnp.float32),
    out_specs=pl.BlockSpec(memory_space=pltpu.SMEM),
)


def kernel(A_rows, A_cols, A_values, b, c, x, Iy):
    rows2d = A_rows.astype(jnp.int32).reshape(NNZ // 128, 128)
    cols = A_cols.astype(jnp.int32)
    xf = x[:, 0]
    acc = _spmv(rows2d, cols, A_values, xf)
    out = _finish(acc.reshape(NC, 512, 128),
                  b.reshape(512, 128),
                  Iy.reshape(512, 128))
    return out[0, 0]
